# baseline (speedup 1.0000x reference)
import jax
import jax.numpy as jnp
from jax.experimental import pallas as pl
from jax.experimental.pallas import tpu as pltpu

_N = 5000
_K = 12544
_H = 1024
_BN = 2560
_BK = 256
_NK = _K // _BK
_NN = 2

_DN = (((1,), (1,)), ((), ()))


def _body(x_ref, w1_ref, b1_ref, w2_ref, b2_ref, wc_ref, bc_ref, wr_ref,
          br_ref, cls_ref, box_ref, acc_ref):
    k = pl.program_id(1)

    @pl.when(k == 0)
    def _init():
        acc_ref[...] = jnp.zeros_like(acc_ref)

    acc_ref[...] += jax.lax.dot_general(
        x_ref[...], w1_ref[...], _DN, preferred_element_type=jnp.float32)

    @pl.when(k == _NK - 1)
    def _finish():
        h1 = jnp.maximum(acc_ref[...] + b1_ref[...], 0.0)
        h2 = jax.lax.dot_general(
            h1, w2_ref[...], _DN, preferred_element_type=jnp.float32)
        h2 = jnp.maximum(h2 + b2_ref[...], 0.0)
        cls_ref[...] = jax.lax.dot_general(
            h2, wc_ref[...], _DN,
            preferred_element_type=jnp.float32) + bc_ref[...]
        box_ref[...] = jax.lax.dot_general(
            h2, wr_ref[...], _DN,
            preferred_element_type=jnp.float32) + br_ref[...]


def kernel(feature_vectors, W1, b1, W2, b2, Wc, bc, Wr, br):
    c1 = Wc.shape[0]
    c4 = Wr.shape[0]
    cls_out, box_out = pl.pallas_call(
        _body,
        grid=(_NN, _NK),
        in_specs=[
            pl.BlockSpec((_BN, _BK), lambda i, k: (i, k)),
            pl.BlockSpec((_H, _BK), lambda i, k: (0, k)),
            pl.BlockSpec((1, _H), lambda i, k: (0, 0)),
            pl.BlockSpec((_H, _H), lambda i, k: (0, 0)),
            pl.BlockSpec((1, _H), lambda i, k: (0, 0)),
            pl.BlockSpec((c1, _H), lambda i, k: (0, 0)),
            pl.BlockSpec((1, c1), lambda i, k: (0, 0)),
            pl.BlockSpec((c4, _H), lambda i, k: (0, 0)),
            pl.BlockSpec((1, c4), lambda i, k: (0, 0)),
        ],
        out_specs=[
            pl.BlockSpec((_BN, c1), lambda i, k: (i, 0)),
            pl.BlockSpec((_BN, c4), lambda i, k: (i, 0)),
        ],
        out_shape=[
            jax.ShapeDtypeStruct((_N, c1), jnp.float32),
            jax.ShapeDtypeStruct((_N, c4), jnp.float32),
        ],
        scratch_shapes=[pltpu.VMEM((_BN, _H), jnp.float32)],
        compiler_params=pltpu.CompilerParams(
            dimension_semantics=("parallel", "arbitrary")),
    )(feature_vectors, W1, b1.reshape(1, -1), W2, b2.reshape(1, -1),
      Wc, bc.reshape(1, -1), Wr, br.reshape(1, -1))
    return (cls_out, box_out)
